# baseline (speedup 1.0000x reference)
import jax
import jax.numpy as jnp
from jax import lax
from jax.experimental import pallas as pl
from jax.experimental.pallas import tpu as pltpu
from jax.experimental.pallas import tpu_sc as plsc

N_NODES = 10000
IN_DIM = 256
OUT_DIM = 256
BATCH = 2048
SLOPE = 0.1

NPAD = 10016
TRASH0 = 2048
NC, NS = 2, 16
NW = NC * NS
CH = 256
NCH = 20
CHB = 256
EPT = CH * NCH
E_PAD = NW * EPT
REG = EPT + CHB
BPT = BATCH // NW
NSL = OUT_DIM // 16

_MESH = plsc.VectorSubcoreMesh(core_axis_name="c", subcore_axis_name="s")


def _tc_body(feat_ref, w_ref, b_ref, a2_ref, emb_ref, al_ref):
    e = jnp.dot(feat_ref[...], w_ref[...], preferred_element_type=jnp.float32)
    e = e + b_ref[...]
    emb_ref[...] = e
    al_ref[...] = jnp.dot(e, a2_ref[...], preferred_element_type=jnp.float32)


def _tc_project(feat, W, b2, a2pad):
    blk = 1000
    return pl.pallas_call(
        _tc_body,
        grid=(N_NODES // blk,),
        in_specs=[
            pl.BlockSpec((blk, IN_DIM), lambda i: (i, 0)),
            pl.BlockSpec((IN_DIM, OUT_DIM), lambda i: (0, 0)),
            pl.BlockSpec((1, OUT_DIM), lambda i: (0, 0)),
            pl.BlockSpec((OUT_DIM, 128), lambda i: (0, 0)),
        ],
        out_specs=[
            pl.BlockSpec((blk, OUT_DIM), lambda i: (i, 0)),
            pl.BlockSpec((blk, 128), lambda i: (i, 0)),
        ],
        out_shape=[
            jax.ShapeDtypeStruct((N_NODES, OUT_DIM), jnp.float32),
            jax.ShapeDtypeStruct((N_NODES, 128), jnp.float32),
        ],
    )(feat, W, b2, a2pad)


def _sc_main_body(emb2_hbm, as_hbm, ad_hbm, src_hbm, dst_hbm, nodes_hbm,
                  res_hbm, den_hbm, inv_hbm,
                  reg_r, reg_v, reg_d, counts_sh,
                  inv_t, nodes_v, srcv, dstv, rv, asv, adv, valv,
                  pend_r, pend_v, pend_d, cbuf,
                  rcv, valcv, dstcv, gix, g, acc2, den_acc,
                  semA, semB, semR, semG):
    cid = lax.axis_index("c")
    sid = lax.axis_index("s")
    wid = sid * NC + cid
    iota = lax.iota(jnp.int32, 16)
    z16 = jnp.zeros((16,), jnp.int32)
    zf16 = jnp.zeros((16,), jnp.float32)
    m_true = iota >= 0

    @pl.when(sid == 0)
    def _build_inv():
        def init_body(j, carry):
            inv_t[pl.ds(j * 16, 16)] = jnp.full((16,), TRASH0, jnp.int32)
            return carry
        lax.fori_loop(0, NPAD // 16, init_body, 0)
        pltpu.sync_copy(nodes_hbm, nodes_v)

        def scat_body(i, carry):
            idx = nodes_v[pl.ds(i * 16, 16)]
            plsc.store_scatter(inv_t, [idx], iota + i * 16)
            return carry
        lax.fori_loop(0, BATCH // 16, scat_body, 0)
        pltpu.sync_copy(inv_t, inv_hbm)

    plsc.subcore_barrier()

    base_e = wid * EPT

    def fetch_sd(ci, buf):
        off = base_e + ci * CH
        pltpu.async_copy(src_hbm.at[pl.ds(off, CH)], srcv.at[buf], semA)
        pltpu.async_copy(dst_hbm.at[pl.ds(off, CH)], dstv.at[buf], semA)

    def drain_sd(buf):
        pltpu.make_async_copy(src_hbm.at[pl.ds(0, CH)], srcv.at[buf],
                              semA).wait()
        pltpu.make_async_copy(dst_hbm.at[pl.ds(0, CH)], dstv.at[buf],
                              semA).wait()

    def fire_alpha(buf):
        pltpu.async_copy(inv_hbm.at[srcv.at[buf]], rv.at[buf], semB)
        pltpu.async_copy(as_hbm.at[srcv.at[buf]], asv.at[buf], semB)
        pltpu.async_copy(ad_hbm.at[dstv.at[buf]], adv.at[buf], semB)

    def drain_alpha(buf):
        pltpu.make_async_copy(inv_hbm.at[srcv.at[buf]], rv.at[buf],
                              semB).wait()
        pltpu.make_async_copy(as_hbm.at[srcv.at[buf]], asv.at[buf],
                              semB).wait()
        pltpu.make_async_copy(ad_hbm.at[dstv.at[buf]], adv.at[buf],
                              semB).wait()

    def compact(buf, cnt):
        for j in range(CH // 16):
            s = pl.ds(j * 16, 16)
            x = asv[buf, s] + adv[buf, s]
            valv[s] = jnp.exp(jnp.maximum(x, SLOPE * x))
        for j in range(CH // 16):
            s = pl.ds(j * 16, 16)
            rj = rv[buf, s]
            m = rj < TRASH0
            plsc.store_compressed(pend_r.at[pl.ds(cnt, 16)], rj, mask=m)
            plsc.store_compressed(pend_v.at[pl.ds(cnt, 16)], valv[s], mask=m)
            plsc.store_compressed(pend_d.at[pl.ds(cnt, 16)], dstv[buf, s],
                                  mask=m)
            cnt = cnt + jnp.sum(m.astype(jnp.int32))
        return cnt

    fetch_sd(0, 0)
    drain_sd(0)
    fire_alpha(0)

    def pair_body(h, cnt):
        fetch_sd(2 * h + 1, 1)
        drain_alpha(0)
        drain_sd(1)
        fire_alpha(1)
        cnt = compact(0, cnt)
        fetch_sd(jnp.minimum(2 * h + 2, NCH - 1), 0)
        drain_alpha(1)
        drain_sd(0)
        fire_alpha(0)
        cnt = compact(1, cnt)
        return cnt

    cnt = lax.fori_loop(0, NCH // 2, pair_body, jnp.int32(0))
    drain_alpha(0)

    for k in range(CHB // 16):
        o = pl.ds(cnt + k * 16, 16)
        plsc.store_compressed(pend_r.at[o], jnp.full((16,), TRASH0,
                                                     jnp.int32), mask=m_true)
        plsc.store_compressed(pend_v.at[o], zf16, mask=m_true)
        plsc.store_compressed(pend_d.at[o], z16, mask=m_true)

    pltpu.sync_copy(pend_r, reg_r.at[sid])
    pltpu.sync_copy(pend_v, reg_v.at[sid])
    pltpu.sync_copy(pend_d, reg_d.at[sid])
    cbuf[pl.ds(0, 16)] = z16 + cnt
    pltpu.sync_copy(cbuf, counts_sh.at[sid])

    plsc.subcore_barrier()

    def zacc_body(i, carry):
        acc2[i, pl.ds(0, 16)] = zf16
        return carry
    lax.fori_loop(0, BATCH, zacc_body, 0)

    def zden_body(i, carry):
        den_acc[pl.ds(i * 16, 16)] = zf16
        return carry
    lax.fori_loop(0, BATCH // 16, zden_body, 0)

    def region_body(w, carry):
        pltpu.sync_copy(counts_sh.at[w], cbuf)
        cnt_w = jnp.sum(cbuf[pl.ds(0, 16)]) // 16
        nb = jnp.maximum((cnt_w + (CHB - 1)) // CHB, 1)

        def fetch_regs(b, buf):
            o = pl.ds(b * CHB, CHB)
            pltpu.async_copy(reg_r.at[w, o], rcv.at[buf], semR)
            pltpu.async_copy(reg_v.at[w, o], valcv.at[buf], semR)
            pltpu.async_copy(reg_d.at[w, o], dstcv.at[buf], semR)

        def drain_regs(buf):
            o = pl.ds(0, CHB)
            pltpu.make_async_copy(reg_r.at[w, o], rcv.at[buf], semR).wait()
            pltpu.make_async_copy(reg_v.at[w, o], valcv.at[buf], semR).wait()
            pltpu.make_async_copy(reg_d.at[w, o], dstcv.at[buf], semR).wait()

        def fire_rows(buf):
            for j in range(CHB // 16):
                s = pl.ds(j * 16, 16)
                gix[buf, s] = dstcv[buf, s] * NSL + sid
            pltpu.async_copy(emb2_hbm.at[gix.at[buf]], g.at[buf], semG)

        def drain_rows():
            pltpu.make_async_copy(emb2_hbm.at[gix.at[0]], g.at[0],
                                  semG).wait()

        def proc(buf):
            @pl.when(w == sid)
            def _den():
                for j in range(CHB // 16):
                    s = pl.ds(j * 16, 16)
                    rj = rcv[buf, s]
                    plsc.addupdate_scatter(den_acc, [rj], valcv[buf, s],
                                           mask=rj < TRASH0)

            def edge_body(h2, c3):
                e0 = 4 * h2
                for u in range(4):
                    e = e0 + u
                    rb = plsc.load_gather(rcv.at[buf], [z16 + e])
                    vb = plsc.load_gather(valcv.at[buf], [z16 + e])
                    row = g[buf, e, pl.ds(0, 16)]
                    plsc.addupdate_scatter(acc2, [rb, iota], row * vb,
                                           mask=rb < TRASH0)
                return c3
            lax.fori_loop(0, CHB // 4, edge_body, 0)

        fetch_regs(0, 0)
        drain_regs(0)
        fire_rows(0)

        def pairb_body(h, c2):
            b1 = 2 * h + 1
            fetch_regs(jnp.minimum(b1, nb - 1), 1)
            drain_rows()
            drain_regs(1)
            fire_rows(1)
            proc(0)
            fetch_regs(jnp.minimum(2 * h + 2, nb - 1), 0)
            drain_rows()
            drain_regs(0)
            fire_rows(0)

            @pl.when(b1 < nb)
            def _p1():
                proc(1)
            return c2

        lax.fori_loop(0, (nb + 1) // 2, pairb_body, 0)
        drain_rows()
        return carry

    lax.fori_loop(0, NS, region_body, 0)

    pltpu.sync_copy(acc2, res_hbm.at[cid, sid])
    pltpu.sync_copy(den_acc, den_hbm.at[cid, sid])


_sc_main = pl.kernel(
    _sc_main_body,
    out_type=[
        jax.ShapeDtypeStruct((NC, NS, BATCH, 16), jnp.float32),
        jax.ShapeDtypeStruct((NC, NS, BATCH), jnp.float32),
        jax.ShapeDtypeStruct((NPAD,), jnp.int32),
    ],
    mesh=_MESH,
    compiler_params=pltpu.CompilerParams(needs_layout_passes=False,
                                         use_tc_tiling_on_sc=False),
    scratch_types=[
        pltpu.VMEM_SHARED((NS, REG), jnp.int32),
        pltpu.VMEM_SHARED((NS, REG), jnp.float32),
        pltpu.VMEM_SHARED((NS, REG), jnp.int32),
        pltpu.VMEM_SHARED((NS, 16), jnp.int32),
        pltpu.VMEM((NPAD,), jnp.int32),
        pltpu.VMEM((BATCH,), jnp.int32),
        pltpu.VMEM((2, CH), jnp.int32),
        pltpu.VMEM((2, CH), jnp.int32),
        pltpu.VMEM((2, CH), jnp.int32),
        pltpu.VMEM((2, CH), jnp.float32),
        pltpu.VMEM((2, CH), jnp.float32),
        pltpu.VMEM((CH,), jnp.float32),
        pltpu.VMEM((REG,), jnp.int32),
        pltpu.VMEM((REG,), jnp.float32),
        pltpu.VMEM((REG,), jnp.int32),
        pltpu.VMEM((16,), jnp.int32),
        pltpu.VMEM((2, CHB), jnp.int32),
        pltpu.VMEM((2, CHB), jnp.float32),
        pltpu.VMEM((2, CHB), jnp.int32),
        pltpu.VMEM((2, CHB), jnp.int32),
        pltpu.VMEM((2, CHB, 16), jnp.float32),
        pltpu.VMEM((BATCH, 16), jnp.float32),
        pltpu.VMEM((BATCH,), jnp.float32),
        pltpu.SemaphoreType.DMA,
        pltpu.SemaphoreType.DMA,
        pltpu.SemaphoreType.DMA,
        pltpu.SemaphoreType.DMA,
    ],
)


def _sc_fin_body(res_hbm, den_hbm, inv_hbm, nodes_hbm,
                 out_hbm, nv, rv, rvt32, gbig, dtmp32, dsum, out2, sem):
    cid = lax.axis_index("c")
    sid = lax.axis_index("s")
    wid = sid * NC + cid
    base = wid * BPT
    z16 = jnp.zeros((16,), jnp.int32)
    zf16 = jnp.zeros((16,), jnp.float32)
    pltpu.sync_copy(nodes_hbm.at[pl.ds(base, BPT)], nv)
    pltpu.sync_copy(inv_hbm.at[nv], rv)

    for slab in range(NC * NS):
        for j in range(BPT // 16):
            s = pl.ds(j * 16, 16)
            rvt32[slab, s] = rv[s] + slab * BATCH

    cps = []
    for slab in range(NC * NS):
        cps.append(pltpu.async_copy(den_hbm.at[rvt32.at[slab]],
                                    dtmp32.at[slab], sem))
        cps.append(pltpu.async_copy(res_hbm.at[rvt32.at[slab]],
                                    gbig.at[slab], sem))
    for cp in cps:
        cp.wait()

    for j in range(BPT // 16):
        s = pl.ds(j * 16, 16)
        dd = zf16
        for slab in range(NC * NS):
            dd = dd + dtmp32[slab, s]
        dsum[s] = jnp.where(dd > 0.0, dd, 1.0)

    for q in range(NSL):
        def row_body(e, carry):
            db = plsc.load_gather(dsum, [z16 + e])
            out2[e, pl.ds(q * 16, 16)] = \
                (gbig[q, e, pl.ds(0, 16)] + gbig[NS + q, e, pl.ds(0, 16)]) / db
            return carry
        lax.fori_loop(0, BPT, row_body, 0)

    pltpu.sync_copy(out2, out_hbm.at[pl.ds(base, BPT)])


_sc_fin = pl.kernel(
    _sc_fin_body,
    out_type=jax.ShapeDtypeStruct((BATCH, OUT_DIM), jnp.float32),
    mesh=_MESH,
    compiler_params=pltpu.CompilerParams(needs_layout_passes=False,
                                         use_tc_tiling_on_sc=False),
    scratch_types=[
        pltpu.VMEM((BPT,), jnp.int32),
        pltpu.VMEM((BPT,), jnp.int32),
        pltpu.VMEM((NC * NS, BPT), jnp.int32),
        pltpu.VMEM((NC * NS, BPT, 16), jnp.float32),
        pltpu.VMEM((NC * NS, BPT), jnp.float32),
        pltpu.VMEM((BPT,), jnp.float32),
        pltpu.VMEM((BPT, OUT_DIM), jnp.float32),
        pltpu.SemaphoreType.DMA,
    ],
)


def kernel(feat, W, b, a, edge_index, nodes):
    feat = feat.astype(jnp.float32)
    W = W.astype(jnp.float32)
    b2 = b.astype(jnp.float32).reshape(1, OUT_DIM)
    a = a.astype(jnp.float32)
    a2 = jnp.concatenate([a[:OUT_DIM], a[OUT_DIM:]], axis=1)
    a2pad = jnp.pad(a2, ((0, 0), (0, 126)))

    emb, alphas = _tc_project(feat, W, b2, a2pad)
    emb2 = emb.reshape(N_NODES * NSL, 16)
    as_t = jnp.concatenate(
        [alphas[:, 0], jnp.full((NPAD - N_NODES,), -1e30, jnp.float32)])
    ad_t = jnp.concatenate(
        [alphas[:, 1], jnp.zeros((NPAD - N_NODES,), jnp.float32)])

    nodes_i = nodes.astype(jnp.int32)
    n_fill = E_PAD - edge_index.shape[0] - BATCH
    src = jnp.concatenate([
        edge_index[:, 0].astype(jnp.int32), nodes_i,
        jnp.full((n_fill,), N_NODES, jnp.int32)])
    dst = jnp.concatenate([
        edge_index[:, 1].astype(jnp.int32), nodes_i,
        jnp.zeros((n_fill,), jnp.int32)])

    res, den, inv = _sc_main(emb2, as_t, ad_t, src, dst, nodes_i)
    res2 = res.reshape(NC * NS * BATCH, 16)
    den2 = den.reshape(NC * NS * BATCH)
    return _sc_fin(res2, den2, inv, nodes_i)


# speedup vs baseline: 1.0564x; 1.0564x over previous
import jax
import jax.numpy as jnp
from jax import lax
from jax.experimental import pallas as pl
from jax.experimental.pallas import tpu as pltpu
from jax.experimental.pallas import tpu_sc as plsc

N_NODES = 10000
IN_DIM = 256
OUT_DIM = 256
BATCH = 2048
SLOPE = 0.1

NPAD = 10016
TRASH0 = 2048
NC, NS = 2, 16
NW = NC * NS
CH = 256
NCH = 20
CHB = 256
EPT = CH * NCH
E_PAD = NW * EPT
REG = EPT + CHB
BPT = BATCH // NW
NSL = OUT_DIM // 16

_MESH = plsc.VectorSubcoreMesh(core_axis_name="c", subcore_axis_name="s")


def _tc_body(feat_ref, w_ref, b_ref, a2_ref, emb_ref, al_ref):
    e = jnp.dot(feat_ref[...], w_ref[...], preferred_element_type=jnp.float32)
    e = e + b_ref[...]
    emb_ref[...] = e
    al_ref[...] = jnp.dot(e, a2_ref[...], preferred_element_type=jnp.float32)


def _tc_project(feat, W, b2, a2pad):
    blk = 1000
    return pl.pallas_call(
        _tc_body,
        grid=(N_NODES // blk,),
        in_specs=[
            pl.BlockSpec((blk, IN_DIM), lambda i: (i, 0)),
            pl.BlockSpec((IN_DIM, OUT_DIM), lambda i: (0, 0)),
            pl.BlockSpec((1, OUT_DIM), lambda i: (0, 0)),
            pl.BlockSpec((OUT_DIM, 128), lambda i: (0, 0)),
        ],
        out_specs=[
            pl.BlockSpec((blk, OUT_DIM), lambda i: (i, 0)),
            pl.BlockSpec((blk, 128), lambda i: (i, 0)),
        ],
        out_shape=[
            jax.ShapeDtypeStruct((N_NODES, OUT_DIM), jnp.float32),
            jax.ShapeDtypeStruct((N_NODES, 128), jnp.float32),
        ],
    )(feat, W, b2, a2pad)


def _sc_main_body(emb2_hbm, as_hbm, ad_hbm, src_hbm, dst_hbm, nodes_hbm,
                  res_hbm, den_hbm, inv_hbm,
                  reg_r, reg_v, reg_d, counts_sh,
                  inv_t, nodes_v, srcv, dstv, rv, asv, adv, valv,
                  pend_r, pend_v, pend_d, cbuf,
                  rcv, valcv, dstcv, gix, g, acc2, den_acc,
                  semA, semB, semR, semG):
    cid = lax.axis_index("c")
    sid = lax.axis_index("s")
    wid = sid * NC + cid
    iota = lax.iota(jnp.int32, 16)
    z16 = jnp.zeros((16,), jnp.int32)
    zf16 = jnp.zeros((16,), jnp.float32)
    m_true = iota >= 0

    @pl.when(sid == 0)
    def _build_inv():
        def init_body(j, carry):
            inv_t[pl.ds(j * 16, 16)] = jnp.full((16,), TRASH0, jnp.int32)
            return carry
        lax.fori_loop(0, NPAD // 16, init_body, 0)
        pltpu.sync_copy(nodes_hbm, nodes_v)

        def scat_body(i, carry):
            idx = nodes_v[pl.ds(i * 16, 16)]
            plsc.store_scatter(inv_t, [idx], iota + i * 16)
            return carry
        lax.fori_loop(0, BATCH // 16, scat_body, 0)
        pltpu.sync_copy(inv_t, inv_hbm)

    plsc.subcore_barrier()

    base_e = wid * EPT

    def fetch_sd(ci, buf):
        off = base_e + ci * CH
        pltpu.async_copy(src_hbm.at[pl.ds(off, CH)], srcv.at[buf], semA)
        pltpu.async_copy(dst_hbm.at[pl.ds(off, CH)], dstv.at[buf], semA)

    def drain_sd(buf):
        pltpu.make_async_copy(src_hbm.at[pl.ds(0, CH)], srcv.at[buf],
                              semA).wait()
        pltpu.make_async_copy(dst_hbm.at[pl.ds(0, CH)], dstv.at[buf],
                              semA).wait()

    def fire_alpha(buf):
        pltpu.async_copy(inv_hbm.at[srcv.at[buf]], rv.at[buf], semB)
        pltpu.async_copy(as_hbm.at[srcv.at[buf]], asv.at[buf], semB)
        pltpu.async_copy(ad_hbm.at[dstv.at[buf]], adv.at[buf], semB)

    def drain_alpha(buf):
        pltpu.make_async_copy(inv_hbm.at[srcv.at[buf]], rv.at[buf],
                              semB).wait()
        pltpu.make_async_copy(as_hbm.at[srcv.at[buf]], asv.at[buf],
                              semB).wait()
        pltpu.make_async_copy(ad_hbm.at[dstv.at[buf]], adv.at[buf],
                              semB).wait()

    def compact(buf, cnt):
        for j in range(CH // 16):
            s = pl.ds(j * 16, 16)
            x = asv[buf, s] + adv[buf, s]
            valv[s] = jnp.exp(jnp.maximum(x, SLOPE * x))
        for j in range(CH // 16):
            s = pl.ds(j * 16, 16)
            rj = rv[buf, s]
            m = rj < TRASH0
            plsc.store_compressed(pend_r.at[pl.ds(cnt, 16)], rj, mask=m)
            plsc.store_compressed(pend_v.at[pl.ds(cnt, 16)], valv[s], mask=m)
            plsc.store_compressed(pend_d.at[pl.ds(cnt, 16)], dstv[buf, s],
                                  mask=m)
            cnt = cnt + jnp.sum(m.astype(jnp.int32))
        return cnt

    fetch_sd(0, 0)
    drain_sd(0)
    fire_alpha(0)

    def pair_body(h, cnt):
        fetch_sd(2 * h + 1, 1)
        drain_alpha(0)
        drain_sd(1)
        fire_alpha(1)
        cnt = compact(0, cnt)
        fetch_sd(jnp.minimum(2 * h + 2, NCH - 1), 0)
        drain_alpha(1)
        drain_sd(0)
        fire_alpha(0)
        cnt = compact(1, cnt)
        return cnt

    cnt = lax.fori_loop(0, NCH // 2, pair_body, jnp.int32(0))
    drain_alpha(0)

    for k in range(CHB // 16):
        o = pl.ds(cnt + k * 16, 16)
        plsc.store_compressed(pend_r.at[o], jnp.full((16,), TRASH0,
                                                     jnp.int32), mask=m_true)
        plsc.store_compressed(pend_v.at[o], zf16, mask=m_true)
        plsc.store_compressed(pend_d.at[o], z16, mask=m_true)

    pltpu.sync_copy(pend_r, reg_r.at[sid])
    pltpu.sync_copy(pend_v, reg_v.at[sid])
    pltpu.sync_copy(pend_d, reg_d.at[sid])
    cbuf[pl.ds(0, 16)] = z16 + cnt
    pltpu.sync_copy(cbuf, counts_sh.at[sid])

    plsc.subcore_barrier()

    def zacc_body(i, carry):
        acc2[i, pl.ds(0, 16)] = zf16
        return carry
    lax.fori_loop(0, BATCH, zacc_body, 0)

    def zden_body(i, carry):
        den_acc[pl.ds(i * 16, 16)] = zf16
        return carry
    lax.fori_loop(0, BATCH // 16, zden_body, 0)

    def region_body(w, carry):
        pltpu.sync_copy(counts_sh.at[w], cbuf)
        cnt_w = jnp.sum(cbuf[pl.ds(0, 16)]) // 16
        nb = jnp.maximum((cnt_w + (CHB - 1)) // CHB, 1)

        def fetch_regs(b, buf):
            o = pl.ds(b * CHB, CHB)
            pltpu.async_copy(reg_r.at[w, o], rcv.at[buf], semR)
            pltpu.async_copy(reg_v.at[w, o], valcv.at[buf], semR)
            pltpu.async_copy(reg_d.at[w, o], dstcv.at[buf], semR)

        def drain_regs(buf):
            o = pl.ds(0, CHB)
            pltpu.make_async_copy(reg_r.at[w, o], rcv.at[buf], semR).wait()
            pltpu.make_async_copy(reg_v.at[w, o], valcv.at[buf], semR).wait()
            pltpu.make_async_copy(reg_d.at[w, o], dstcv.at[buf], semR).wait()

        def fire_rows(buf):
            for j in range(CHB // 16):
                s = pl.ds(j * 16, 16)
                gix[buf, s] = dstcv[buf, s] * NSL + sid
            pltpu.async_copy(emb2_hbm.at[gix.at[buf]], g.at[buf], semG)

        def drain_rows():
            pltpu.make_async_copy(emb2_hbm.at[gix.at[0]], g.at[0],
                                  semG).wait()

        def proc(buf):
            @pl.when(w == sid)
            def _den():
                for j in range(CHB // 16):
                    s = pl.ds(j * 16, 16)
                    rj = rcv[buf, s]
                    plsc.addupdate_scatter(den_acc, [rj], valcv[buf, s],
                                           mask=rj < TRASH0)

            def edge_body(h2, c3):
                e = 2 * h2
                rb = plsc.load_gather(rcv.at[buf], [z16 + e])
                vb = plsc.load_gather(valcv.at[buf], [z16 + e])
                rb2 = plsc.load_gather(rcv.at[buf], [z16 + (e + 1)])
                vb2 = plsc.load_gather(valcv.at[buf], [z16 + (e + 1)])
                row = g[buf, e, pl.ds(0, 16)]
                row2 = g[buf, e + 1, pl.ds(0, 16)]
                plsc.addupdate_scatter(acc2, [rb, iota], row * vb,
                                       mask=rb < TRASH0)
                plsc.addupdate_scatter(acc2, [rb2, iota], row2 * vb2,
                                       mask=rb2 < TRASH0)
                return c3
            lax.fori_loop(0, CHB // 2, edge_body, 0)

        fetch_regs(0, 0)
        drain_regs(0)
        fire_rows(0)

        def pairb_body(h, c2):
            b1 = 2 * h + 1
            fetch_regs(jnp.minimum(b1, nb - 1), 1)
            drain_rows()
            drain_regs(1)
            fire_rows(1)
            proc(0)
            fetch_regs(jnp.minimum(2 * h + 2, nb - 1), 0)
            drain_rows()
            drain_regs(0)
            fire_rows(0)

            @pl.when(b1 < nb)
            def _p1():
                proc(1)
            return c2

        lax.fori_loop(0, (nb + 1) // 2, pairb_body, 0)
        drain_rows()
        return carry

    lax.fori_loop(0, NS, region_body, 0)

    pltpu.sync_copy(acc2, res_hbm.at[cid, sid])
    pltpu.sync_copy(den_acc, den_hbm.at[cid, sid])


_sc_main = pl.kernel(
    _sc_main_body,
    out_type=[
        jax.ShapeDtypeStruct((NC, NS, BATCH, 16), jnp.float32),
        jax.ShapeDtypeStruct((NC, NS, BATCH), jnp.float32),
        jax.ShapeDtypeStruct((NPAD,), jnp.int32),
    ],
    mesh=_MESH,
    compiler_params=pltpu.CompilerParams(needs_layout_passes=False,
                                         use_tc_tiling_on_sc=False),
    scratch_types=[
        pltpu.VMEM_SHARED((NS, REG), jnp.int32),
        pltpu.VMEM_SHARED((NS, REG), jnp.float32),
        pltpu.VMEM_SHARED((NS, REG), jnp.int32),
        pltpu.VMEM_SHARED((NS, 16), jnp.int32),
        pltpu.VMEM((NPAD,), jnp.int32),
        pltpu.VMEM((BATCH,), jnp.int32),
        pltpu.VMEM((2, CH), jnp.int32),
        pltpu.VMEM((2, CH), jnp.int32),
        pltpu.VMEM((2, CH), jnp.int32),
        pltpu.VMEM((2, CH), jnp.float32),
        pltpu.VMEM((2, CH), jnp.float32),
        pltpu.VMEM((CH,), jnp.float32),
        pltpu.VMEM((REG,), jnp.int32),
        pltpu.VMEM((REG,), jnp.float32),
        pltpu.VMEM((REG,), jnp.int32),
        pltpu.VMEM((16,), jnp.int32),
        pltpu.VMEM((2, CHB), jnp.int32),
        pltpu.VMEM((2, CHB), jnp.float32),
        pltpu.VMEM((2, CHB), jnp.int32),
        pltpu.VMEM((2, CHB), jnp.int32),
        pltpu.VMEM((2, CHB, 16), jnp.float32),
        pltpu.VMEM((BATCH, 16), jnp.float32),
        pltpu.VMEM((BATCH,), jnp.float32),
        pltpu.SemaphoreType.DMA,
        pltpu.SemaphoreType.DMA,
        pltpu.SemaphoreType.DMA,
        pltpu.SemaphoreType.DMA,
    ],
)


def _sc_fin_body(res_hbm, den_hbm, inv_hbm, nodes_hbm,
                 out_hbm, nv, rv, rvt32, gbig, dtmp32, dsum, out2, sem):
    cid = lax.axis_index("c")
    sid = lax.axis_index("s")
    wid = sid * NC + cid
    base = wid * BPT
    z16 = jnp.zeros((16,), jnp.int32)
    zf16 = jnp.zeros((16,), jnp.float32)
    pltpu.sync_copy(nodes_hbm.at[pl.ds(base, BPT)], nv)
    pltpu.sync_copy(inv_hbm.at[nv], rv)

    for slab in range(NC * NS):
        for j in range(BPT // 16):
            s = pl.ds(j * 16, 16)
            rvt32[slab, s] = rv[s] + slab * BATCH

    cps = []
    for slab in range(NC * NS):
        cps.append(pltpu.async_copy(den_hbm.at[rvt32.at[slab]],
                                    dtmp32.at[slab], sem))
        cps.append(pltpu.async_copy(res_hbm.at[rvt32.at[slab]],
                                    gbig.at[slab], sem))
    for cp in cps:
        cp.wait()

    for j in range(BPT // 16):
        s = pl.ds(j * 16, 16)
        dd = zf16
        for slab in range(NC * NS):
            dd = dd + dtmp32[slab, s]
        dsum[s] = jnp.where(dd > 0.0, dd, 1.0)

    for q in range(NSL):
        def row_body(e, carry):
            db = plsc.load_gather(dsum, [z16 + e])
            out2[e, pl.ds(q * 16, 16)] = \
                (gbig[q, e, pl.ds(0, 16)] + gbig[NS + q, e, pl.ds(0, 16)]) / db
            return carry
        lax.fori_loop(0, BPT, row_body, 0)

    pltpu.sync_copy(out2, out_hbm.at[pl.ds(base, BPT)])


_sc_fin = pl.kernel(
    _sc_fin_body,
    out_type=jax.ShapeDtypeStruct((BATCH, OUT_DIM), jnp.float32),
    mesh=_MESH,
    compiler_params=pltpu.CompilerParams(needs_layout_passes=False,
                                         use_tc_tiling_on_sc=False),
    scratch_types=[
        pltpu.VMEM((BPT,), jnp.int32),
        pltpu.VMEM((BPT,), jnp.int32),
        pltpu.VMEM((NC * NS, BPT), jnp.int32),
        pltpu.VMEM((NC * NS, BPT, 16), jnp.float32),
        pltpu.VMEM((NC * NS, BPT), jnp.float32),
        pltpu.VMEM((BPT,), jnp.float32),
        pltpu.VMEM((BPT, OUT_DIM), jnp.float32),
        pltpu.SemaphoreType.DMA,
    ],
)


def kernel(feat, W, b, a, edge_index, nodes):
    feat = feat.astype(jnp.float32)
    W = W.astype(jnp.float32)
    b2 = b.astype(jnp.float32).reshape(1, OUT_DIM)
    a = a.astype(jnp.float32)
    a2 = jnp.concatenate([a[:OUT_DIM], a[OUT_DIM:]], axis=1)
    a2pad = jnp.pad(a2, ((0, 0), (0, 126)))

    emb, alphas = _tc_project(feat, W, b2, a2pad)
    emb2 = emb.reshape(N_NODES * NSL, 16)
    as_t = jnp.concatenate(
        [alphas[:, 0], jnp.full((NPAD - N_NODES,), -1e30, jnp.float32)])
    ad_t = jnp.concatenate(
        [alphas[:, 1], jnp.zeros((NPAD - N_NODES,), jnp.float32)])

    nodes_i = nodes.astype(jnp.int32)
    n_fill = E_PAD - edge_index.shape[0] - BATCH
    src = jnp.concatenate([
        edge_index[:, 0].astype(jnp.int32), nodes_i,
        jnp.full((n_fill,), N_NODES, jnp.int32)])
    dst = jnp.concatenate([
        edge_index[:, 1].astype(jnp.int32), nodes_i,
        jnp.zeros((n_fill,), jnp.int32)])

    res, den, inv = _sc_main(emb2, as_t, ad_t, src, dst, nodes_i)
    res2 = res.reshape(NC * NS * BATCH, 16)
    den2 = den.reshape(NC * NS * BATCH)
    return _sc_fin(res2, den2, inv, nodes_i)
